# baseline (speedup 1.0000x reference)
import functools
from itertools import combinations

import jax
import jax.numpy as jnp
import numpy as np
from jax import lax
from jax.experimental import pallas as pl
from jax.experimental.pallas import tpu as pltpu
from jax.experimental.pallas import tpu_sc as plsc

B = 4096
F = 26
V = 100000
D = 16
H = 128
NP = 325
FD = F * D

NC = 2
NS = 16
NW = NC * NS
CHUNK = (B * F) // NW
NSTREAM = CHUNK // 128

_PAIRS = np.array(list(combinations(range(F), 2)), dtype=np.int32)

BB = B // NW


def _sc_gather(tab_hbm, xi_hbm, out_hbm, idx_v, rows_v, sem):
    wid = lax.axis_index("s") * NC + lax.axis_index("c")
    b0 = wid * BB
    pltpu.sync_copy(xi_hbm.at[wid], idx_v)
    gathers = [
        pltpu.async_copy(
            tab_hbm.at[j].at[idx_v.at[j]],
            rows_v.at[pl.ds(j * BB, BB)],
            sem,
        )
        for j in range(F)
    ]
    for cp in gathers:
        cp.wait()
    stores = [
        pltpu.async_copy(
            rows_v.at[pl.ds(j * BB, BB)],
            out_hbm.at[pl.ds(b0, BB), pl.ds(j * D, D)],
            sem,
        )
        for j in range(F)
    ]
    for cp in stores:
        cp.wait()


@functools.partial(jax.jit, static_argnames=())
def _gather_rows(tables, xi_r):
    mesh = plsc.VectorSubcoreMesh(
        core_axis_name="c", subcore_axis_name="s", num_cores=NC, num_subcores=NS
    )
    return pl.kernel(
        _sc_gather,
        out_type=jax.ShapeDtypeStruct((B, FD), jnp.float32),
        mesh=mesh,
        compiler_params=pltpu.CompilerParams(use_tc_tiling_on_sc=False),
        scratch_types=[
            pltpu.VMEM((F, BB), jnp.int32),
            pltpu.VMEM((F * BB, D), jnp.float32),
            pltpu.SemaphoreType.DMA,
        ],
    )(tables, xi_r)


def _tc_body(eraw_ref, xvr_ref, k_ref, xv_ref, w1_ref, b1_ref, w2h_ref, b2_ref,
             out_ref):
    e = eraw_ref[...] * xvr_ref[...]
    y = jnp.dot(e, k_ref[...], preferred_element_type=jnp.float32)
    quad = 0.5 * jnp.sum(e * y, axis=1)
    h = jnp.maximum(
        jnp.dot(xv_ref[...], w1_ref[...], preferred_element_type=jnp.float32)
        + b1_ref[...], 0.0)
    dense = jnp.sum(h * w2h_ref[...], axis=1)
    out_ref[...] = (quad + dense + b2_ref[0, 0])[:, None]


def _tc_combine(eraw2, xvr, K, xv, W1, b1r, w2h, b2r):
    bm = 512
    grid = B // bm
    return pl.pallas_call(
        _tc_body,
        grid=(grid,),
        in_specs=[
            pl.BlockSpec((bm, FD), lambda i: (i, 0)),
            pl.BlockSpec((bm, FD), lambda i: (i, 0)),
            pl.BlockSpec((FD, FD), lambda i: (0, 0)),
            pl.BlockSpec((bm, F), lambda i: (i, 0)),
            pl.BlockSpec((F, H), lambda i: (0, 0)),
            pl.BlockSpec((1, H), lambda i: (0, 0)),
            pl.BlockSpec((1, H), lambda i: (0, 0)),
            pl.BlockSpec((1, 1), lambda i: (0, 0)),
        ],
        out_specs=pl.BlockSpec((bm, 1), lambda i: (i, 0)),
        out_shape=jax.ShapeDtypeStruct((B, 1), jnp.float32),
    )(eraw2, xvr, K, xv, W1, b1r, w2h, b2r)


def kernel(xv, xi, tables, W1, b1, W2, b2):
    xi32 = xi.astype(jnp.int32)
    xi_r = jnp.transpose(xi32).reshape(F, NW, BB).transpose(1, 0, 2)

    eraw2 = _gather_rows(tables, xi_r)

    pi = jnp.asarray(_PAIRS[:, 0])
    pj = jnp.asarray(_PAIRS[:, 1])
    w_q = W2[:NP, 0]
    A = jnp.zeros((F, F), jnp.float32).at[pi, pj].set(w_q)
    A = A + A.T
    K = jnp.einsum("fg,de->fdge", A, jnp.eye(D, dtype=jnp.float32))
    K = K.reshape(FD, FD)

    xvr = jnp.repeat(xv, D, axis=1)
    b1r = b1.reshape(1, H)
    w2h = W2[NP:, 0].reshape(1, H)
    b2r = b2.reshape(1, 1)

    return _tc_combine(eraw2, xvr, K, xv, W1, b1r, w2h, b2r)


# speedup vs baseline: 3.3530x; 3.3530x over previous
import functools
from itertools import combinations

import jax
import jax.numpy as jnp
import numpy as np
from jax import lax
from jax.experimental import pallas as pl
from jax.experimental.pallas import tpu as pltpu
from jax.experimental.pallas import tpu_sc as plsc

B = 4096
F = 26
V = 100000
D = 16
H = 128
NP = 325
FD = F * D

NC = 2
NS = 16
NW = NC * NS
BB = B // NW

_PAIRS = np.array(list(combinations(range(F), 2)), dtype=np.int32)


def _sc_gather(tab_hbm, xi_hbm, out_hbm, idx_v, rows_v, sem):
    wid = lax.axis_index("s") * NC + lax.axis_index("c")
    b0 = wid * BB
    pltpu.sync_copy(xi_hbm.at[wid], idx_v)

    def fire(f, c):
        for d in range(D):
            pltpu.async_copy(
                tab_hbm.at[f * D + d].at[idx_v.at[f]],
                rows_v.at[f * D + d],
                sem,
            )
        return c

    lax.fori_loop(0, F, fire, 0)

    def drain(f, c):
        for d in range(D):
            pltpu.make_async_copy(
                tab_hbm.at[f * D + d].at[idx_v.at[f]],
                rows_v.at[f * D + d],
                sem,
            ).wait()
        return c

    lax.fori_loop(0, F, drain, 0)
    pltpu.sync_copy(rows_v, out_hbm.at[:, pl.ds(b0, BB)])


@functools.partial(jax.jit, static_argnames=())
def _gather_planes(tabT, xi_r):
    mesh = plsc.VectorSubcoreMesh(
        core_axis_name="c", subcore_axis_name="s", num_cores=NC, num_subcores=NS
    )
    return pl.kernel(
        _sc_gather,
        out_type=jax.ShapeDtypeStruct((FD, B), jnp.float32),
        mesh=mesh,
        compiler_params=pltpu.CompilerParams(use_tc_tiling_on_sc=False),
        scratch_types=[
            pltpu.VMEM((F, BB), jnp.int32),
            pltpu.VMEM((FD, BB), jnp.float32),
            pltpu.SemaphoreType.DMA,
        ],
    )(tabT, xi_r)


def _tc_body(eT_ref, xvT_ref, k_ref, w1_ref, b1_ref, w2h_ref, b2_ref, out_ref):
    xvT = xvT_ref[...]
    xvrT = jnp.reshape(
        jnp.broadcast_to(xvT[:, None, :], (F, D, xvT.shape[1])),
        (FD, xvT.shape[1]))
    e = eT_ref[...] * xvrT
    y = jnp.dot(k_ref[...], e, preferred_element_type=jnp.float32)
    quad = 0.5 * jnp.sum(e * y, axis=0)
    hT = jnp.maximum(
        lax.dot_general(w1_ref[...], xvT, (((0,), (0,)), ((), ())),
                        preferred_element_type=jnp.float32)
        + b1_ref[...], 0.0)
    dense = jnp.sum(hT * w2h_ref[...], axis=0)
    out_ref[...] = (quad + dense + b2_ref[0, 0])[None, :]


def _tc_combine(eT, xvT, K, W1, b1c, w2hc, b2r):
    bn = 512
    grid = B // bn
    return pl.pallas_call(
        _tc_body,
        grid=(grid,),
        in_specs=[
            pl.BlockSpec((FD, bn), lambda i: (0, i)),
            pl.BlockSpec((F, bn), lambda i: (0, i)),
            pl.BlockSpec((FD, FD), lambda i: (0, 0)),
            pl.BlockSpec((F, H), lambda i: (0, 0)),
            pl.BlockSpec((H, 1), lambda i: (0, 0)),
            pl.BlockSpec((H, 1), lambda i: (0, 0)),
            pl.BlockSpec((1, 1), lambda i: (0, 0)),
        ],
        out_specs=pl.BlockSpec((1, bn), lambda i: (0, i)),
        out_shape=jax.ShapeDtypeStruct((1, B), jnp.float32),
    )(eT, xvT, K, W1, b1c, w2hc, b2r)


def kernel(xv, xi, tables, W1, b1, W2, b2):
    xi32 = xi.astype(jnp.int32)
    tabT = jnp.transpose(tables, (0, 2, 1)).reshape(FD, V)
    xi_r = jnp.transpose(xi32).reshape(F, NW, BB).transpose(1, 0, 2)

    eT = _gather_planes(tabT, xi_r)

    pi = jnp.asarray(_PAIRS[:, 0])
    pj = jnp.asarray(_PAIRS[:, 1])
    w_q = W2[:NP, 0]
    A = jnp.zeros((F, F), jnp.float32).at[pi, pj].set(w_q)
    A = A + A.T
    K = jnp.einsum("fg,de->fdge", A, jnp.eye(D, dtype=jnp.float32))
    K = K.reshape(FD, FD)

    xvT = jnp.transpose(xv)
    b1c = b1.reshape(H, 1)
    w2hc = W2[NP:, 0].reshape(H, 1)
    b2r = b2.reshape(1, 1)

    outT = _tc_combine(eT, xvT, K, W1, b1c, w2hc, b2r)
    return jnp.transpose(outT)


# speedup vs baseline: 5.3729x; 1.6024x over previous
import functools
from itertools import combinations

import jax
import jax.numpy as jnp
import numpy as np
from jax import lax
from jax.experimental import pallas as pl
from jax.experimental.pallas import tpu as pltpu
from jax.experimental.pallas import tpu_sc as plsc

B = 4096
F = 26
V = 100000
D = 16
H = 128
NP = 325
FD = F * D

NC = 2
NS = 16
NW = NC * NS
BB = B // NW
CH = 2
FC = F // CH
FCD = FC * D

_PAIRS = np.array(list(combinations(range(F), 2)), dtype=np.int32)

VP = 100096
RB = FCD // 8
RPB = 8 * (VP // 128)
ROWS = RB * RPB


def _relayout_body(in_ref, out_ref):
    blk = in_ref[...]
    out_ref[...] = blk.reshape(8, VP // 128, 128).reshape(RPB, 128)


def _tc_relayout(tabT, c):
    return pl.pallas_call(
        _relayout_body,
        grid=(RB,),
        in_specs=[pl.BlockSpec((8, VP), lambda i: (c * RB + i, 0))],
        out_specs=pl.BlockSpec((RPB, 128), lambda i: (i, 0)),
        out_shape=jax.ShapeDtypeStruct((ROWS, 128), jnp.float32),
    )(tabT)


def _sc_gather(tab_hbm, xi_hbm, out_hbm, idx_v, rows_v, sem):
    wid = lax.axis_index("s") * NC + lax.axis_index("c")
    b0 = wid * BB
    pltpu.sync_copy(xi_hbm.at[wid], idx_v)

    def fire(f, c):
        for d in range(D):
            r = f * D + d
            base = (r // 8) * (RPB * 128) + (r % 8) * VP
            pltpu.async_copy(
                tab_hbm.at[pl.ds(base, VP)].at[idx_v.at[f]],
                rows_v.at[r],
                sem,
            )
        return c

    lax.fori_loop(0, FC, fire, 0)

    def drain(f, c):
        for d in range(D):
            r = f * D + d
            base = (r // 8) * (RPB * 128) + (r % 8) * VP
            pltpu.make_async_copy(
                tab_hbm.at[pl.ds(base, VP)].at[idx_v.at[f]],
                rows_v.at[r],
                sem,
            ).wait()
        return c

    lax.fori_loop(0, FC, drain, 0)
    pltpu.sync_copy(rows_v, out_hbm.at[:, pl.ds(b0, BB)])


@functools.partial(jax.jit, static_argnames=())
def _gather_planes(tabT, xi_r):
    mesh = plsc.VectorSubcoreMesh(
        core_axis_name="c", subcore_axis_name="s", num_cores=NC, num_subcores=NS
    )
    return pl.kernel(
        _sc_gather,
        out_type=jax.ShapeDtypeStruct((FCD, B), jnp.float32),
        mesh=mesh,
        compiler_params=pltpu.CompilerParams(use_tc_tiling_on_sc=False),
        scratch_types=[
            pltpu.VMEM((FC, BB), jnp.int32),
            pltpu.VMEM((FCD, BB), jnp.float32),
            pltpu.SemaphoreType.DMA,
        ],
    )(tabT, xi_r)


def _tc_body(eT0_ref, eT1_ref, xvT_ref, k_ref, w1_ref, b1_ref, w2h_ref,
             b2_ref, out_ref):
    xvT = xvT_ref[...]
    xvrT = jnp.reshape(
        jnp.broadcast_to(xvT[:, None, :], (F, D, xvT.shape[1])),
        (FD, xvT.shape[1]))
    eT = jnp.concatenate([eT0_ref[...], eT1_ref[...]], axis=0)
    e = eT * xvrT
    y = jnp.dot(k_ref[...], e, preferred_element_type=jnp.float32)
    quad = 0.5 * jnp.sum(e * y, axis=0)
    hT = jnp.maximum(
        lax.dot_general(w1_ref[...], xvT, (((0,), (0,)), ((), ())),
                        preferred_element_type=jnp.float32)
        + b1_ref[...], 0.0)
    dense = jnp.sum(hT * w2h_ref[...], axis=0)
    out_ref[...] = (quad + dense + b2_ref[0, 0])[None, :]


def _tc_combine(eT0, eT1, xvT, K, W1, b1c, w2hc, b2r):
    bn = 512
    grid = B // bn
    return pl.pallas_call(
        _tc_body,
        grid=(grid,),
        in_specs=[
            pl.BlockSpec((FCD, bn), lambda i: (0, i)),
            pl.BlockSpec((FCD, bn), lambda i: (0, i)),
            pl.BlockSpec((F, bn), lambda i: (0, i)),
            pl.BlockSpec((FD, FD), lambda i: (0, 0)),
            pl.BlockSpec((F, H), lambda i: (0, 0)),
            pl.BlockSpec((H, 1), lambda i: (0, 0)),
            pl.BlockSpec((H, 1), lambda i: (0, 0)),
            pl.BlockSpec((1, 1), lambda i: (0, 0)),
        ],
        out_specs=pl.BlockSpec((1, bn), lambda i: (0, i)),
        out_shape=jax.ShapeDtypeStruct((1, B), jnp.float32),
    )(eT0, eT1, xvT, K, W1, b1c, w2hc, b2r)


def kernel(xv, xi, tables, W1, b1, W2, b2):
    xi32 = xi.astype(jnp.int32)
    tabT = jnp.transpose(tables, (0, 2, 1)).reshape(FD, V)
    xi_r = jnp.transpose(xi32).reshape(F, NW, BB).transpose(1, 0, 2)

    tab1d0 = _tc_relayout(tabT, 0).reshape(ROWS * 128)
    eT0 = _gather_planes(tab1d0, xi_r[:, :FC])
    tab1d1 = _tc_relayout(tabT, 1).reshape(ROWS * 128)
    eT1 = _gather_planes(tab1d1, xi_r[:, FC:])

    pi = jnp.asarray(_PAIRS[:, 0])
    pj = jnp.asarray(_PAIRS[:, 1])
    w_q = W2[:NP, 0]
    A = jnp.zeros((F, F), jnp.float32).at[pi, pj].set(w_q)
    A = A + A.T
    K = jnp.einsum("fg,de->fdge", A, jnp.eye(D, dtype=jnp.float32))
    K = K.reshape(FD, FD)

    xvT = jnp.transpose(xv)
    b1c = b1.reshape(H, 1)
    w2hc = W2[NP:, 0].reshape(H, 1)
    b2r = b2.reshape(1, 1)

    outT = _tc_combine(eT0, eT1, xvT, K, W1, b1c, w2hc, b2r)
    return jnp.transpose(outT)
